# baseline (speedup 1.0000x reference)
import functools

import jax
import jax.numpy as jnp
from jax import lax
from jax.experimental import pallas as pl
from jax.experimental.pallas import tpu as pltpu
from jax.experimental.pallas import tpu_sc as plsc

N = 10000
E = 320000
D = 128

NC = 2
NS = 16
NW = NC * NS

NP = 10240
RPT = NP // NS
EW = E // NW
CH = 80
NCHUNK = EW // CH
NB = 3

_MESH = plsc.VectorSubcoreMesh(
    core_axis_name="c", subcore_axis_name="s", num_cores=NC, num_subcores=NS)


def _sc_agg_body(with_deg, h, src, dst2, *args):
    if with_deg:
        (out, deg_out, idx_d, s0, s1, s2, r0b, r1b, r2b, ones, zeros1, acc, acc1,
         semd, is0, is1, is2, gs0, gs1, gs2, ss0, ss1, ss2) = args
    else:
        (out, idx_d, s0, s1, s2, r0b, r1b, r2b, acc,
         semd, is0, is1, is2, gs0, gs1, gs2, ss0, ss1, ss2) = args
    cid = lax.axis_index("c")
    sid = lax.axis_index("s")
    wid = sid * NC + cid
    base = wid * EW

    sbufs = [s0, s1, s2]
    rbufs = [r0b, r1b, r2b]
    isem = [is0, is1, is2]
    gsem = [gs0, gs1, gs2]
    ssem = [ss0, ss1, ss2]

    pltpu.async_copy(dst2.at[wid], idx_d, semd)

    zc = D // 16

    def fill_z(i, _):
        r0b[i // zc, pl.ds((i % zc) * 16, 16)] = jnp.zeros((16,), jnp.float32)
        return 0

    lax.fori_loop(0, CH * zc, fill_z, 0)
    row0 = sid * RPT
    for i in range(RPT // CH):
        pltpu.sync_copy(r0b, acc.at[pl.ds(row0 + i * CH, CH)])

    if with_deg:
        def fill_ones(i, _):
            ones[pl.ds(i * 16, 16)] = jnp.ones((16,), jnp.float32)
            return 0

        lax.fori_loop(0, CH // 16, fill_ones, 0)

        def fill_z1(i, _):
            zeros1[pl.ds(i * 16, 16)] = jnp.zeros((16,), jnp.float32)
            return 0

        lax.fori_loop(0, RPT // 16, fill_z1, 0)
        pltpu.sync_copy(zeros1, acc1.at[pl.ds(row0, RPT)])

    pltpu.make_async_copy(dst2.at[wid], idx_d, semd).wait()
    plsc.subcore_barrier()

    def iload(k, b):
        pltpu.async_copy(src.at[pl.ds(base + k * CH, CH)], sbufs[b], isem[b])

    def iload_wait(k, b):
        pltpu.make_async_copy(src.at[pl.ds(base + k * CH, CH)], sbufs[b],
                              isem[b]).wait()

    def gath(b):
        pltpu.async_copy(h.at[sbufs[b]], rbufs[b], gsem[b])

    def gath_wait(b):
        pltpu.make_async_copy(h.at[sbufs[b]], rbufs[b], gsem[b]).wait()

    def scat(k, b):
        pltpu.async_copy(rbufs[b], acc.at[idx_d.at[k]], ssem[b], add=True)
        if with_deg:
            pltpu.async_copy(ones, acc1.at[idx_d.at[k]], semd, add=True)

    def scat_wait(k, b):
        pltpu.make_async_copy(rbufs[b], acc.at[idx_d.at[k]], ssem[b]).wait()

    iload(0, 0)
    iload(1, 1)
    iload_wait(0, 0)
    gath(0)

    def body3(j, _):
        for t in range(NB):
            k = j * NB + t
            b = t
            bp1 = (t + 1) % NB
            bp2 = (t + 2) % NB
            @pl.when(jnp.logical_and(k + 2 < NCHUNK, k >= 1))
            def _():
                scat_wait(k - 1, bp2)

            @pl.when(k + 2 < NCHUNK)
            def _():
                iload(k + 2, bp2)

            @pl.when(k + 1 < NCHUNK)
            def _():
                iload_wait(k + 1, bp1)
                gath(bp1)

            @pl.when(k < NCHUNK)
            def _():
                gath_wait(b)
                scat(k, b)
        return 0

    lax.fori_loop(0, (NCHUNK + NB - 1) // NB, body3, 0)

    for t in range(NB):
        k = NCHUNK - NB + t
        scat_wait(k, k % NB)

    if with_deg:
        def drain_ones(k, _):
            pltpu.make_async_copy(ones, acc1.at[idx_d.at[0]], semd).wait()
            return 0

        lax.fori_loop(0, NCHUNK, drain_ones, 0)
    plsc.subcore_barrier()

    pltpu.sync_copy(acc.at[pl.ds(row0, RPT)], out.at[cid, pl.ds(row0, RPT)])
    if with_deg:
        pltpu.sync_copy(acc1.at[pl.ds(row0, RPT)], deg_out.at[cid, pl.ds(row0, RPT)])


def _sc_agg(h, src, dst2, with_deg):
    out_type = [jax.ShapeDtypeStruct((NC, NP, D), jnp.float32)]
    scratch = [
        pltpu.VMEM((NCHUNK, CH), jnp.int32),
        pltpu.VMEM((CH,), jnp.int32),
        pltpu.VMEM((CH,), jnp.int32),
        pltpu.VMEM((CH,), jnp.int32),
        pltpu.VMEM((CH, D), jnp.float32),
        pltpu.VMEM((CH, D), jnp.float32),
        pltpu.VMEM((CH, D), jnp.float32),
    ]
    nsem = 10
    if with_deg:
        out_type.append(jax.ShapeDtypeStruct((NC, NP), jnp.float32))
        scratch.append(pltpu.VMEM((CH,), jnp.float32))
        scratch.append(pltpu.VMEM((RPT,), jnp.float32))
    scratch.append(pltpu.VMEM_SHARED((NP, D), jnp.float32))
    if with_deg:
        scratch.append(pltpu.VMEM_SHARED((NP,), jnp.float32))
    scratch += [pltpu.SemaphoreType.DMA] * nsem
    kfn = pl.kernel(
        functools.partial(_sc_agg_body, with_deg),
        out_type=out_type,
        mesh=_MESH,
        scratch_types=scratch,
    )
    return kfn(h, src, dst2)


BR = 2000


def _hs_body(h_ref, ws_ref, b_ref, out_ref):
    out_ref[...] = jnp.dot(h_ref[...], ws_ref[...],
                           preferred_element_type=jnp.float32) + b_ref[...]


def _hs(h, Ws, b):
    Dout = Ws.shape[1]
    return pl.pallas_call(
        _hs_body,
        grid=(N // BR,),
        in_specs=[
            pl.BlockSpec((BR, D), lambda i: (i, 0)),
            pl.BlockSpec((D, Dout), lambda i: (0, 0)),
            pl.BlockSpec((1, Dout), lambda i: (0, 0)),
        ],
        out_specs=pl.BlockSpec((BR, Dout), lambda i: (i, 0)),
        out_shape=jax.ShapeDtypeStruct((N, Dout), jnp.float32),
    )(h, Ws, b.reshape(1, -1))


def _combine_body(relu, hs_ref, parts_ref, inv_ref, wn_ref, out_ref):
    agg = (parts_ref[0] + parts_ref[1]) * inv_ref[...]
    o = hs_ref[...] + jnp.dot(agg, wn_ref[...], preferred_element_type=jnp.float32)
    if relu:
        o = jnp.maximum(o, 0.0)
    out_ref[...] = o


def _combine(hs, parts, inv, Wn, relu):
    Dout = Wn.shape[1]
    return pl.pallas_call(
        functools.partial(_combine_body, relu),
        grid=(N // BR,),
        in_specs=[
            pl.BlockSpec((BR, Dout), lambda i: (i, 0)),
            pl.BlockSpec((NC, BR, D), lambda i: (0, i, 0)),
            pl.BlockSpec((BR, 1), lambda i: (i, 0)),
            pl.BlockSpec((D, Dout), lambda i: (0, 0)),
        ],
        out_specs=pl.BlockSpec((BR, Dout), lambda i: (i, 0)),
        out_shape=jax.ShapeDtypeStruct((N, Dout), jnp.float32),
    )(hs, parts, inv, Wn)


def kernel(x, edge_index, Ws0, Wn0, b0, Ws1, Wn1, b1, Ws2, Wn2, b2):
    src = edge_index[0]
    dst = edge_index[1]
    dst2 = dst.reshape(NW, NCHUNK, CH)

    h = x
    inv = None
    for l, (Ws, Wn, b) in enumerate([(Ws0, Wn0, b0), (Ws1, Wn1, b1), (Ws2, Wn2, b2)]):
        if l == 0:
            parts, deg_parts = _sc_agg(h, src, dst2, with_deg=True)
            deg = deg_parts[0, :N] + deg_parts[1, :N]
            inv = (1.0 / jnp.clip(deg, 1.0, None)).reshape(N, 1)
        else:
            (parts,) = _sc_agg(h, src, dst2, with_deg=False)
        hs = _hs(h, Ws, b)
        h = _combine(hs, parts, inv, Wn, relu=(l != 2))
    return h
